# Initial kernel scaffold; baseline (speedup 1.0000x reference)
import jax
import jax.numpy as jnp
from jax.experimental import pallas as pl


def kernel(x, edge_index, batch, exp_embedding, exp_bias, W1, b1, W2, b2, lin2_W, lin2_b):
    raise NotImplementedError("write your pallas kernel here")


# speedup vs baseline: 50.4862x; 50.4862x over previous
import functools

import jax
import jax.numpy as jnp
from jax import lax
from jax.experimental import pallas as pl
from jax.experimental.pallas import tpu as pltpu
from jax.experimental.pallas import tpu_sc as plsc

NUM_GENES = 10000
FEAT = 16
BATCH = 10
N = NUM_GENES * BATCH
E = 3_200_000

NCORE = 2
NSUB = 16
NTILE = NCORE * NSUB

NP = 100_096
NODES_PER_TILE = NP // NSUB
PT_ROWS = 784
EP = NTILE * PT_ROWS * 128
EROWS = EP // 128
CHUNK_ROWS = 8
NCHUNK = PT_ROWS // CHUNK_ROWS
STAGE = 368
NSTAGE = NODES_PER_TILE // STAGE

_mesh = plsc.VectorSubcoreMesh(
    core_axis_name="c", subcore_axis_name="s",
    num_cores=NCORE, num_subcores=NSUB)

_sc_params = pltpu.CompilerParams(use_tc_tiling_on_sc=False)


@functools.partial(
    pl.kernel,
    out_type=jax.ShapeDtypeStruct((NCORE * NP,), jnp.float32),
    mesh=_mesh,
    scratch_types=[
        pltpu.VMEM_SHARED((NP,), jnp.float32),
        pltpu.VMEM((CHUNK_ROWS, 128), jnp.int32),
        pltpu.VMEM((128,), jnp.float32),
        pltpu.VMEM((NODES_PER_TILE,), jnp.float32),
    ],
    compiler_params=_sc_params,
)
def _sc_degree(dst_hbm, deg_out, deg_sh, idx_v, ones_v, stage_v):
    c = lax.axis_index("c")
    s = lax.axis_index("s")
    w = c * NSUB + s

    @pl.loop(0, 128 // 16)
    def _(i):
        ones_v[pl.ds(i * 16, 16)] = jnp.ones((16,), jnp.float32)

    @pl.loop(0, NODES_PER_TILE // 16)
    def _(i):
        stage_v[pl.ds(i * 16, 16)] = jnp.zeros((16,), jnp.float32)

    base = s * NODES_PER_TILE
    pltpu.sync_copy(stage_v, deg_sh.at[pl.ds(base, NODES_PER_TILE)])
    plsc.subcore_barrier()

    @pl.loop(0, NCHUNK)
    def _(g):
        row0 = w * PT_ROWS + g * CHUNK_ROWS
        pltpu.sync_copy(dst_hbm.at[pl.ds(row0, CHUNK_ROWS)], idx_v)
        for j in range(CHUNK_ROWS):
            pltpu.sync_copy(ones_v, deg_sh.at[idx_v.at[j]], add=True)

    plsc.subcore_barrier()
    pltpu.sync_copy(deg_sh.at[pl.ds(base, NODES_PER_TILE)], stage_v)
    pltpu.sync_copy(stage_v, deg_out.at[pl.ds(c * NP + base, NODES_PER_TILE)])


@functools.partial(
    pl.kernel,
    out_type=jax.ShapeDtypeStruct((NCORE * NP, FEAT), jnp.float32),
    mesh=_mesh,
    scratch_types=[
        pltpu.VMEM_SHARED((NP, FEAT), jnp.float32),
        pltpu.VMEM((CHUNK_ROWS, 128), jnp.int32),
        pltpu.VMEM((CHUNK_ROWS, 128), jnp.int32),
        pltpu.VMEM((CHUNK_ROWS * 128, FEAT), jnp.float32),
        pltpu.SemaphoreType.DMA,
    ],
    compiler_params=_sc_params,
)
def _sc_scatter(src_hbm, dst_hbm, table_hbm, acc_out,
                acc_sh, sidx, didx, rows_v, sem):
    c = lax.axis_index("c")
    s = lax.axis_index("s")
    w = c * NSUB + s

    @pl.loop(0, STAGE)
    def _(i):
        rows_v[i] = jnp.zeros((FEAT,), jnp.float32)

    base = s * NODES_PER_TILE

    @pl.loop(0, NSTAGE)
    def _(k):
        pltpu.sync_copy(rows_v.at[pl.ds(0, STAGE)],
                        acc_sh.at[pl.ds(base + k * STAGE, STAGE)])
    plsc.subcore_barrier()

    @pl.loop(0, NCHUNK)
    def _(g):
        row0 = w * PT_ROWS + g * CHUNK_ROWS
        pltpu.sync_copy(src_hbm.at[pl.ds(row0, CHUNK_ROWS)], sidx)
        pltpu.sync_copy(dst_hbm.at[pl.ds(row0, CHUNK_ROWS)], didx)
        copies = [
            pltpu.async_copy(table_hbm.at[sidx.at[j]],
                             rows_v.at[pl.ds(j * 128, 128)], sem)
            for j in range(CHUNK_ROWS)
        ]
        for d in copies:
            d.wait()
        for j in range(CHUNK_ROWS):
            pltpu.sync_copy(rows_v.at[pl.ds(j * 128, 128)],
                            acc_sh.at[didx.at[j]], add=True)

    plsc.subcore_barrier()

    @pl.loop(0, NSTAGE)
    def _(k):
        pltpu.sync_copy(acc_sh.at[pl.ds(base + k * STAGE, STAGE)],
                        rows_v.at[pl.ds(0, STAGE)])
        pltpu.sync_copy(rows_v.at[pl.ds(0, STAGE)],
                        acc_out.at[pl.ds(c * NP + base + k * STAGE, STAGE)])


def _tc_prep_body(x_ref, emb_ref, bias_ref, w1_ref, degp_ref,
                  table_ref, dinv_ref):
    emb = emb_ref[...] * x_ref[...] + bias_ref[...]
    hw = jnp.dot(emb, w1_ref[...], preferred_element_type=jnp.float32)
    dp = degp_ref[...]
    dinv = lax.rsqrt(dp[:, 0] + dp[:, 1] + 1.0)
    table_ref[...] = hw * dinv[:, None]
    dinv_ref[...] = dinv[:, None]


_tc_prep = pl.pallas_call(
    _tc_prep_body,
    grid=(BATCH,),
    in_specs=[
        pl.BlockSpec((NUM_GENES, 1), lambda i: (i, 0)),
        pl.BlockSpec((NUM_GENES, FEAT), lambda i: (0, 0)),
        pl.BlockSpec((NUM_GENES, 1), lambda i: (0, 0)),
        pl.BlockSpec((FEAT, FEAT), lambda i: (0, 0)),
        pl.BlockSpec((NUM_GENES, 2), lambda i: (i, 0)),
    ],
    out_specs=[
        pl.BlockSpec((NUM_GENES, FEAT), lambda i: (i, 0)),
        pl.BlockSpec((NUM_GENES, 1), lambda i: (i, 0)),
    ],
    out_shape=[
        jax.ShapeDtypeStruct((N, FEAT), jnp.float32),
        jax.ShapeDtypeStruct((N, 1), jnp.float32),
    ],
)


def _tc_mid_body(acc_ref, table_ref, dinv_ref, b_ref, w_ref, out_ref):
    a = acc_ref[...]
    dv = dinv_ref[...]
    h = jnp.tanh(dv * (a[0] + a[1] + table_ref[...]) + b_ref[...])
    hw = jnp.dot(h, w_ref[...], preferred_element_type=jnp.float32)
    out_ref[...] = hw * dv


_tc_mid = pl.pallas_call(
    _tc_mid_body,
    grid=(BATCH,),
    in_specs=[
        pl.BlockSpec((2, NUM_GENES, FEAT), lambda i: (0, i, 0)),
        pl.BlockSpec((NUM_GENES, FEAT), lambda i: (i, 0)),
        pl.BlockSpec((NUM_GENES, 1), lambda i: (i, 0)),
        pl.BlockSpec((1, FEAT), lambda i: (0, 0)),
        pl.BlockSpec((FEAT, FEAT), lambda i: (0, 0)),
    ],
    out_specs=pl.BlockSpec((NUM_GENES, FEAT), lambda i: (i, 0)),
    out_shape=jax.ShapeDtypeStruct((N, FEAT), jnp.float32),
)


def _tc_final_body(acc_ref, table_ref, dinv_ref, b_ref, w_ref, lb_ref,
                   out_ref):
    a = acc_ref[...]
    dv = dinv_ref[...]
    h = jnp.tanh(dv * (a[0] + a[1] + table_ref[...]) + b_ref[...])
    pooled = jnp.sum(h, axis=0, keepdims=True) * (1.0 / NUM_GENES)
    r = lax.dot_general(pooled, w_ref[...], (((1,), (1,)), ((), ())),
                        preferred_element_type=jnp.float32)
    out_ref[...] = (r + lb_ref[...])[None]


_tc_final = pl.pallas_call(
    _tc_final_body,
    grid=(BATCH,),
    in_specs=[
        pl.BlockSpec((2, NUM_GENES, FEAT), lambda i: (0, i, 0)),
        pl.BlockSpec((NUM_GENES, FEAT), lambda i: (i, 0)),
        pl.BlockSpec((NUM_GENES, 1), lambda i: (i, 0)),
        pl.BlockSpec((1, FEAT), lambda i: (0, 0)),
        pl.BlockSpec((FEAT, FEAT), lambda i: (0, 0)),
        pl.BlockSpec((1, FEAT), lambda i: (0, 0)),
    ],
    out_specs=pl.BlockSpec((1, 1, FEAT), lambda i: (i, 0, 0)),
    out_shape=jax.ShapeDtypeStruct((BATCH, 1, FEAT), jnp.float32),
)


@jax.jit
def kernel(x, edge_index, batch, exp_embedding, exp_bias,
           W1, b1, W2, b2, lin2_W, lin2_b):
    src = edge_index[0]
    dst = edge_index[1]
    srcP = jnp.concatenate(
        [src, jnp.zeros((EP - E,), jnp.int32)]).reshape(EROWS, 128)
    dstP = jnp.concatenate(
        [dst, jnp.full((EP - E,), N, jnp.int32)]).reshape(EROWS, 128)

    deg_p = _sc_degree(dstP).reshape(NCORE, NP)[:, :N].T

    table1, dinv = _tc_prep(x, exp_embedding, exp_bias, W1, deg_p)

    acc1 = _sc_scatter(srcP, dstP, table1).reshape(NCORE, NP, FEAT)[:, :N]
    table2 = _tc_mid(acc1, table1, dinv, b1.reshape(1, FEAT), W2)

    acc2 = _sc_scatter(srcP, dstP, table2).reshape(NCORE, NP, FEAT)[:, :N]
    out = _tc_final(acc2, table2, dinv, b2.reshape(1, FEAT), lin2_W,
                    lin2_b.reshape(1, FEAT))
    return out.reshape(BATCH, FEAT)


# speedup vs baseline: 62.5855x; 1.2397x over previous
import functools

import jax
import jax.numpy as jnp
from jax import lax
from jax.experimental import pallas as pl
from jax.experimental.pallas import tpu as pltpu
from jax.experimental.pallas import tpu_sc as plsc

NUM_GENES = 10000
FEAT = 16
BATCH = 10
N = NUM_GENES * BATCH
E = 3_200_000

NCORE = 2
NSUB = 16
NTILE = NCORE * NSUB

NP = 100_096
NODES_PER_TILE = NP // NSUB
PT_ROWS = 784
EP = NTILE * PT_ROWS * 128
EROWS = EP // 128
CHUNK_ROWS = 4
CHUNK_EDGES = CHUNK_ROWS * 128
NCHUNK = PT_ROWS // CHUNK_ROWS
DCHUNK_ROWS = 8
DNCHUNK = PT_ROWS // DCHUNK_ROWS
STAGE = 368
NSTAGE = NODES_PER_TILE // STAGE

_mesh = plsc.VectorSubcoreMesh(
    core_axis_name="c", subcore_axis_name="s",
    num_cores=NCORE, num_subcores=NSUB)

_sc_params = pltpu.CompilerParams(use_tc_tiling_on_sc=False)


@functools.partial(
    pl.kernel,
    out_type=jax.ShapeDtypeStruct((NCORE * NP,), jnp.float32),
    mesh=_mesh,
    scratch_types=[
        pltpu.VMEM_SHARED((NP,), jnp.float32),
        pltpu.VMEM((DCHUNK_ROWS, 128), jnp.int32),
        pltpu.VMEM((DCHUNK_ROWS, 128), jnp.int32),
        pltpu.VMEM((128,), jnp.float32),
        pltpu.VMEM((NODES_PER_TILE,), jnp.float32),
        pltpu.SemaphoreType.DMA,
        pltpu.SemaphoreType.DMA,
    ],
    compiler_params=_sc_params,
)
def _sc_degree(dst_hbm, deg_out, deg_sh, idx0, idx1, ones_v, stage_v,
               sem0, sem1):
    c = lax.axis_index("c")
    s = lax.axis_index("s")
    w = c * NSUB + s
    idx = (idx0, idx1)
    sem = (sem0, sem1)

    @pl.loop(0, 128 // 16)
    def _(i):
        ones_v[pl.ds(i * 16, 16)] = jnp.ones((16,), jnp.float32)

    @pl.loop(0, NODES_PER_TILE // 16)
    def _(i):
        stage_v[pl.ds(i * 16, 16)] = jnp.zeros((16,), jnp.float32)

    base = s * NODES_PER_TILE
    pltpu.sync_copy(stage_v, deg_sh.at[pl.ds(base, NODES_PER_TILE)])
    plsc.subcore_barrier()

    dwords = DCHUNK_ROWS * 128

    def dbody(g, b, drain_s):
        if drain_s:
            pltpu.make_async_copy(deg_out.at[pl.ds(0, dwords)],
                                  stage_v.at[pl.ds(0, dwords)],
                                  sem[b]).wait()
        row0 = w * PT_ROWS + g * DCHUNK_ROWS
        pltpu.sync_copy(dst_hbm.at[pl.ds(row0, DCHUNK_ROWS)], idx[b])
        for j in range(DCHUNK_ROWS):
            pltpu.async_copy(ones_v, deg_sh.at[idx[b].at[j]], sem[b],
                             add=True)

    dbody(0, 0, False)
    dbody(1, 1, False)

    @pl.loop(1, DNCHUNK // 2)
    def _(i2):
        g0 = 2 * i2
        dbody(g0, 0, True)
        dbody(g0 + 1, 1, True)

    for b in range(2):
        pltpu.make_async_copy(deg_out.at[pl.ds(0, dwords)],
                              stage_v.at[pl.ds(0, dwords)], sem[b]).wait()

    plsc.subcore_barrier()
    pltpu.sync_copy(deg_sh.at[pl.ds(base, NODES_PER_TILE)], stage_v)
    pltpu.sync_copy(stage_v, deg_out.at[pl.ds(c * NP + base, NODES_PER_TILE)])


@functools.partial(
    pl.kernel,
    out_type=jax.ShapeDtypeStruct((NCORE * NP, FEAT), jnp.float32),
    mesh=_mesh,
    scratch_types=[
        pltpu.VMEM_SHARED((NP, FEAT), jnp.float32),
        pltpu.VMEM((CHUNK_ROWS, 128), jnp.int32),
        pltpu.VMEM((CHUNK_ROWS, 128), jnp.int32),
        pltpu.VMEM((CHUNK_ROWS, 128), jnp.int32),
        pltpu.VMEM((CHUNK_ROWS, 128), jnp.int32),
        pltpu.VMEM((CHUNK_EDGES, FEAT), jnp.float32),
        pltpu.VMEM((CHUNK_EDGES, FEAT), jnp.float32),
        pltpu.SemaphoreType.DMA,
        pltpu.SemaphoreType.DMA,
        pltpu.SemaphoreType.DMA,
        pltpu.SemaphoreType.DMA,
    ],
    compiler_params=_sc_params,
)
def _sc_scatter(src_hbm, dst_hbm, table_hbm, acc_out,
                acc_sh, sidx0, sidx1, didx0, didx1, rows0, rows1,
                semg0, semg1, sems0, sems1):
    c = lax.axis_index("c")
    s = lax.axis_index("s")
    w = c * NSUB + s
    sidx = (sidx0, sidx1)
    didx = (didx0, didx1)
    rows = (rows0, rows1)
    semg = (semg0, semg1)
    sems = (sems0, sems1)

    @pl.loop(0, STAGE)
    def _(i):
        rows0[i] = jnp.zeros((FEAT,), jnp.float32)

    base = s * NODES_PER_TILE

    @pl.loop(0, NSTAGE)
    def _(k):
        pltpu.sync_copy(rows0.at[pl.ds(0, STAGE)],
                        acc_sh.at[pl.ds(base + k * STAGE, STAGE)])
    plsc.subcore_barrier()

    def load_and_gather(g, b):
        row0 = w * PT_ROWS + g * CHUNK_ROWS
        pltpu.sync_copy(src_hbm.at[pl.ds(row0, CHUNK_ROWS)], sidx[b])
        pltpu.sync_copy(dst_hbm.at[pl.ds(row0, CHUNK_ROWS)], didx[b])
        for j in range(CHUNK_ROWS):
            pltpu.async_copy(table_hbm.at[sidx[b].at[j]],
                             rows[b].at[pl.ds(j * 128, 128)], semg[b])

    def drain(sem, b):
        pltpu.make_async_copy(table_hbm.at[pl.ds(0, CHUNK_EDGES)],
                              rows[b], sem).wait()

    def body(g, b, drain_prev_s, prefetch):
        nb = 1 - b
        if drain_prev_s:
            drain(sems[nb], nb)
        if prefetch:
            load_and_gather(g + 1, nb)
        drain(semg[b], b)
        for j in range(CHUNK_ROWS):
            pltpu.async_copy(rows[b].at[pl.ds(j * 128, 128)],
                             acc_sh.at[didx[b].at[j]], sems[b], add=True)

    load_and_gather(0, 0)
    body(0, 0, False, True)
    body(1, 1, True, True)

    @pl.loop(1, NCHUNK // 2 - 1)
    def _(i2):
        g0 = 2 * i2
        body(g0, 0, True, True)
        body(g0 + 1, 1, True, True)

    body(NCHUNK - 2, 0, True, True)
    body(NCHUNK - 1, 1, True, False)
    drain(sems[1], 1)

    plsc.subcore_barrier()

    @pl.loop(0, NSTAGE)
    def _(k):
        pltpu.sync_copy(acc_sh.at[pl.ds(base + k * STAGE, STAGE)],
                        rows0.at[pl.ds(0, STAGE)])
        pltpu.sync_copy(rows0.at[pl.ds(0, STAGE)],
                        acc_out.at[pl.ds(c * NP + base + k * STAGE, STAGE)])


def _tc_prep_body(x_ref, emb_ref, bias_ref, w1_ref, degp_ref,
                  table_ref, dinv_ref):
    emb = emb_ref[...] * x_ref[...] + bias_ref[...]
    hw = jnp.dot(emb, w1_ref[...], preferred_element_type=jnp.float32)
    dp = degp_ref[...]
    dinv = lax.rsqrt(dp[:, 0] + dp[:, 1] + 1.0)
    table_ref[...] = hw * dinv[:, None]
    dinv_ref[...] = dinv[:, None]


_tc_prep = pl.pallas_call(
    _tc_prep_body,
    grid=(BATCH,),
    in_specs=[
        pl.BlockSpec((NUM_GENES, 1), lambda i: (i, 0)),
        pl.BlockSpec((NUM_GENES, FEAT), lambda i: (0, 0)),
        pl.BlockSpec((NUM_GENES, 1), lambda i: (0, 0)),
        pl.BlockSpec((FEAT, FEAT), lambda i: (0, 0)),
        pl.BlockSpec((NUM_GENES, 2), lambda i: (i, 0)),
    ],
    out_specs=[
        pl.BlockSpec((NUM_GENES, FEAT), lambda i: (i, 0)),
        pl.BlockSpec((NUM_GENES, 1), lambda i: (i, 0)),
    ],
    out_shape=[
        jax.ShapeDtypeStruct((N, FEAT), jnp.float32),
        jax.ShapeDtypeStruct((N, 1), jnp.float32),
    ],
)


def _tc_mid_body(acc_ref, table_ref, dinv_ref, b_ref, w_ref, out_ref):
    a = acc_ref[...]
    dv = dinv_ref[...]
    h = jnp.tanh(dv * (a[0] + a[1] + table_ref[...]) + b_ref[...])
    hw = jnp.dot(h, w_ref[...], preferred_element_type=jnp.float32)
    out_ref[...] = hw * dv


_tc_mid = pl.pallas_call(
    _tc_mid_body,
    grid=(BATCH,),
    in_specs=[
        pl.BlockSpec((2, NUM_GENES, FEAT), lambda i: (0, i, 0)),
        pl.BlockSpec((NUM_GENES, FEAT), lambda i: (i, 0)),
        pl.BlockSpec((NUM_GENES, 1), lambda i: (i, 0)),
        pl.BlockSpec((1, FEAT), lambda i: (0, 0)),
        pl.BlockSpec((FEAT, FEAT), lambda i: (0, 0)),
    ],
    out_specs=pl.BlockSpec((NUM_GENES, FEAT), lambda i: (i, 0)),
    out_shape=jax.ShapeDtypeStruct((N, FEAT), jnp.float32),
)


def _tc_final_body(acc_ref, table_ref, dinv_ref, b_ref, w_ref, lb_ref,
                   out_ref):
    a = acc_ref[...]
    dv = dinv_ref[...]
    h = jnp.tanh(dv * (a[0] + a[1] + table_ref[...]) + b_ref[...])
    pooled = jnp.sum(h, axis=0, keepdims=True) * (1.0 / NUM_GENES)
    r = lax.dot_general(pooled, w_ref[...], (((1,), (1,)), ((), ())),
                        preferred_element_type=jnp.float32)
    out_ref[...] = (r + lb_ref[...])[None]


_tc_final = pl.pallas_call(
    _tc_final_body,
    grid=(BATCH,),
    in_specs=[
        pl.BlockSpec((2, NUM_GENES, FEAT), lambda i: (0, i, 0)),
        pl.BlockSpec((NUM_GENES, FEAT), lambda i: (i, 0)),
        pl.BlockSpec((NUM_GENES, 1), lambda i: (i, 0)),
        pl.BlockSpec((1, FEAT), lambda i: (0, 0)),
        pl.BlockSpec((FEAT, FEAT), lambda i: (0, 0)),
        pl.BlockSpec((1, FEAT), lambda i: (0, 0)),
    ],
    out_specs=pl.BlockSpec((1, 1, FEAT), lambda i: (i, 0, 0)),
    out_shape=jax.ShapeDtypeStruct((BATCH, 1, FEAT), jnp.float32),
)


@jax.jit
def kernel(x, edge_index, batch, exp_embedding, exp_bias,
           W1, b1, W2, b2, lin2_W, lin2_b):
    src = edge_index[0]
    dst = edge_index[1]
    srcP = jnp.concatenate(
        [src, jnp.zeros((EP - E,), jnp.int32)]).reshape(EROWS, 128)
    dstP = jnp.concatenate(
        [dst, jnp.full((EP - E,), N, jnp.int32)]).reshape(EROWS, 128)

    deg_p = _sc_degree(dstP).reshape(NCORE, NP)[:, :N].T

    table1, dinv = _tc_prep(x, exp_embedding, exp_bias, W1, deg_p)

    acc1 = _sc_scatter(srcP, dstP, table1).reshape(NCORE, NP, FEAT)[:, :N]
    table2 = _tc_mid(acc1, table1, dinv, b1.reshape(1, FEAT), W2)

    acc2 = _sc_scatter(srcP, dstP, table2).reshape(NCORE, NP, FEAT)[:, :N]
    out = _tc_final(acc2, table2, dinv, b2.reshape(1, FEAT), lin2_W,
                    lin2_b.reshape(1, FEAT))
    return out.reshape(BATCH, FEAT)
